# baseline (speedup 1.0000x reference)
import jax
import jax.numpy as jnp
from jax import lax
from jax.experimental import pallas as pl
from jax.experimental.pallas import tpu as pltpu
from jax.experimental.pallas import tpu_sc as plsc

_BATCH = 16384
_FIELDS = 26
_DIM = 16
_NC = 2
_NS = 16
_NW = _NC * _NS
_ROWS_PER_W = _BATCH // _NW
_CHUNK = 64
_NCHUNK = _ROWS_PER_W // _CHUNK


def _fm_body(idx_hbm, val_hbm, emb_hbm, w1_hbm, bias_hbm, out_hbm,
             idx2_v, idxf_v, val_v, emb_v, w_v, out_v, bias_v, sem):
    wid = lax.axis_index("s") * _NC + lax.axis_index("c")
    pltpu.sync_copy(bias_hbm, bias_v)
    lane = lax.iota(jnp.int32, 16)
    bias_vec = bias_v[...]
    himask = lane >= (2 * 16 - _FIELDS)

    def _stage(c, p):
        row0 = wid * _ROWS_PER_W + c * _CHUNK
        ng = _CHUNK * _FIELDS // 128
        pltpu.sync_copy(idx_hbm.at[pl.ds(row0, _CHUNK), :], idx2_v.at[p])
        pltpu.sync_copy(val_hbm.at[pl.ds(row0, _CHUNK), :], val_v.at[p])

        def _compact(r, carry):
            jb = r * _FIELDS
            idxf_v[p, pl.ds(jb, 16)] = idx2_v[p, r, pl.ds(0, 16)]
            idxf_v[p, pl.ds(jb + _FIELDS - 16, 16)] = (
                idx2_v[p, r, pl.ds(_FIELDS - 16, 16)])
            return carry

        lax.fori_loop(0, _CHUNK, _compact, jnp.int32(0))
        copies = []
        for g in range(ng):
            isl = idxf_v.at[p, pl.ds(g * 128, 128)]
            copies.append(pltpu.async_copy(
                emb_hbm.at[isl], emb_v.at[p, pl.ds(g * 128, 128)], sem))
            copies.append(pltpu.async_copy(
                w1_hbm.at[isl], w_v.at[p, pl.ds(g * 128, 128)], sem))
        return copies

    def _compute(c, p):
        row0 = wid * _ROWS_PER_W + c * _CHUNK

        def _group(g, carry):
            def _row(rr, acc):
                r = g * 16 + rr
                xv0 = val_v[p, r, pl.ds(0, 16)]
                xv1 = val_v[p, r, pl.ds(_FIELDS - 16, 16)]
                jb = r * _FIELDS
                wv0 = w_v[p, pl.ds(jb, 16)]
                wv1 = w_v[p, pl.ds(jb + _FIELDS - 16, 16)]
                fo_vec = xv0 * wv0 + jnp.where(himask, xv1 * wv1, 0.0)
                s = jnp.zeros((16,), jnp.float32)
                sq = jnp.zeros((16,), jnp.float32)
                jbase = r * _FIELDS
                for f in range(_FIELDS):
                    x = xv0[f] if f < 16 else xv1[f - (_FIELDS - 16)]
                    xb = jnp.full((16,), x, jnp.float32)
                    row = emb_v[p, jbase + f, :]
                    ev = row * xb
                    s = s + ev
                    sq = sq + ev * ev
                red = jnp.sum(fo_vec + 0.5 * (s * s - sq))
                return jnp.where(lane == rr, red, acc)

            acc = lax.fori_loop(0, 16, _row, jnp.zeros((16,), jnp.float32))
            logit = bias_vec + acc
            out_v[pl.ds(g * 16, 16)] = 1.0 / (1.0 + jnp.exp(-logit))
            return carry

        lax.fori_loop(0, _CHUNK // 16, _group, jnp.int32(0))
        pltpu.sync_copy(out_v, out_hbm.at[pl.ds(row0, _CHUNK)])

    inflight = _stage(0, 0)
    for c in range(_NCHUNK):
        for cp in inflight:
            cp.wait()
        if c + 1 < _NCHUNK:
            nxt = _stage(c + 1, (c + 1) % 2)
        else:
            nxt = []
        _compute(c, c % 2)
        inflight = nxt


_TSUB = 256


_TMAIN = 15872


def _transpose_body(in_hbm, tail_hbm, out_hbm, vin, vtail, vout,
                    sem_i, sem_o):
    c = pl.program_id(0)

    @pl.when(c < 63)
    def _main():
        pltpu.async_copy(
            in_hbm.at[:, pl.ds(c * _TMAIN, _TMAIN)], vin, sem_i).wait()

        def _one(s, carry):
            o = pl.multiple_of(s * _TSUB, _TSUB)
            xs = vin[:, pl.ds(o, _TSUB)]
            vout[pl.ds(o, _TSUB), :] = jnp.swapaxes(xs, 0, 1)
            return carry

        lax.fori_loop(0, _TMAIN // _TSUB, _one, jnp.int32(0))
        pltpu.async_copy(
            vout, out_hbm.at[pl.ds(c * _TMAIN, _TMAIN)], sem_o).wait()

    @pl.when(c == 63)
    def _tail():
        pltpu.async_copy(tail_hbm, vtail, sem_i).wait()
        zt = jnp.swapaxes(vtail[...], 0, 1)
        vout[pl.ds(0, 64), :] = zt[0:64, :]
        pltpu.async_copy(
            vout.at[pl.ds(0, 64), :],
            out_hbm.at[pl.ds(63 * _TMAIN, 64)], sem_o).wait()


def _relayout_emb(emb_t):
    n_rows = emb_t.shape[1]
    ncut = n_rows - 64
    tail = jnp.pad(jax.lax.slice(emb_t, (0, ncut), (_DIM, n_rows)),
                   ((0, 0), (0, 64)))
    return pl.pallas_call(
        _transpose_body,
        out_shape=jax.ShapeDtypeStruct((n_rows, _DIM), jnp.float32),
        grid=(64,),
        in_specs=[pl.BlockSpec(memory_space=pl.ANY),
                  pl.BlockSpec(memory_space=pl.ANY)],
        out_specs=pl.BlockSpec(memory_space=pl.ANY),
        scratch_shapes=[
            pltpu.VMEM((_DIM, _TMAIN), jnp.float32),
            pltpu.VMEM((_DIM, 128), jnp.float32),
            pltpu.VMEM((_TMAIN, _DIM), jnp.float32),
            pltpu.SemaphoreType.DMA,
            pltpu.SemaphoreType.DMA,
        ],
    )(emb_t, tail)


@jax.jit
def _fm_sc(feat_index, feat_value, emb_table, w1, bias_vec):
    emb_lin = _relayout_emb(emb_table.T)
    mesh = plsc.VectorSubcoreMesh(core_axis_name="c", subcore_axis_name="s")
    return pl.kernel(
        _fm_body,
        out_type=jax.ShapeDtypeStruct((_BATCH,), jnp.float32),
        mesh=mesh,
        compiler_params=pltpu.CompilerParams(
            needs_layout_passes=False, use_tc_tiling_on_sc=False),
        scratch_types=[
            pltpu.VMEM((2, _CHUNK, _FIELDS), jnp.int32),
            pltpu.VMEM((2, _CHUNK * _FIELDS), jnp.int32),
            pltpu.VMEM((2, _CHUNK, _FIELDS), jnp.float32),
            pltpu.VMEM((2, _CHUNK * _FIELDS, _DIM), jnp.float32),
            pltpu.VMEM((2, _CHUNK * _FIELDS), jnp.float32),
            pltpu.VMEM((_CHUNK,), jnp.float32),
            pltpu.VMEM((16,), jnp.float32),
            pltpu.SemaphoreType.DMA,
        ],
    )(feat_index, feat_value, emb_lin, w1, bias_vec)


def kernel(feat_index, feat_value, emb_table, w1, bias):
    bias_vec = jnp.broadcast_to(jnp.asarray(bias, jnp.float32), (16,))
    return _fm_sc(feat_index, feat_value, emb_table, w1.reshape(-1), bias_vec)


# speedup vs baseline: 2.1938x; 2.1938x over previous
import jax
import jax.numpy as jnp
from jax import lax
from jax.experimental import pallas as pl
from jax.experimental.pallas import tpu as pltpu
from jax.experimental.pallas import tpu_sc as plsc

_BATCH = 16384
_FIELDS = 26
_DIM = 16
_NC = 2
_NS = 16
_NW = _NC * _NS
_ROWS_PER_W = _BATCH // _NW
_CHUNK = 64
_NCHUNK = _ROWS_PER_W // _CHUNK
_IPC = _CHUNK * _FIELDS
_GS = 128
_NG = _IPC // _GS


def _fm_body(idx_hbm, val_hbm, emb_hbm, w1_hbm, bias_hbm, out_hbm,
             idx_v, val_v, emb_v, w_v, out_v, bias_v, sem):
    wid = lax.axis_index("s") * _NC + lax.axis_index("c")
    pltpu.sync_copy(bias_hbm, bias_v)
    lane = lax.iota(jnp.int32, 16)
    bias_vec = bias_v[...]
    himask = lane >= (2 * 16 - _FIELDS)

    def _stage(c, p):
        off = (wid * _ROWS_PER_W + c * _CHUNK) * _FIELDS
        pltpu.sync_copy(idx_hbm.at[pl.ds(off, _IPC)], idx_v.at[p])
        pltpu.sync_copy(val_hbm.at[pl.ds(off, _IPC)], val_v.at[p])
        copies = []
        for g in range(_NG):
            isl = idx_v.at[p, pl.ds(g * _GS, _GS)]
            copies.append(pltpu.async_copy(
                emb_hbm.at[isl], emb_v.at[p, pl.ds(g * _GS, _GS)], sem))
            copies.append(pltpu.async_copy(
                w1_hbm.at[isl], w_v.at[p, pl.ds(g * _GS, _GS)], sem))
        return copies

    def _compute(c, p):
        row0 = wid * _ROWS_PER_W + c * _CHUNK

        def _group(g, carry):
            def _row(rr, acc):
                jbase = (g * 16 + rr) * _FIELDS
                xv0 = val_v[p, pl.ds(jbase, 16)]
                xv1 = val_v[p, pl.ds(jbase + _FIELDS - 16, 16)]
                wv0 = w_v[p, pl.ds(jbase, 16)]
                wv1 = w_v[p, pl.ds(jbase + _FIELDS - 16, 16)]
                fo_vec = xv0 * wv0 + jnp.where(himask, xv1 * wv1, 0.0)
                s = jnp.zeros((16,), jnp.float32)
                sq = jnp.zeros((16,), jnp.float32)
                for f in range(_FIELDS):
                    x = xv0[f] if f < 16 else xv1[f - (_FIELDS - 16)]
                    xb = jnp.full((16,), x, jnp.float32)
                    row = emb_v[p, jbase + f, :]
                    ev = row * xb
                    s = s + ev
                    sq = sq + ev * ev
                red = jnp.sum(fo_vec + 0.5 * (s * s - sq))
                return jnp.where(lane == rr, red, acc)

            acc = lax.fori_loop(0, 16, _row, jnp.zeros((16,), jnp.float32))
            logit = bias_vec + acc
            out_v[pl.ds(g * 16, 16)] = 1.0 / (1.0 + jnp.exp(-logit))
            return carry

        lax.fori_loop(0, _CHUNK // 16, _group, jnp.int32(0))
        pltpu.sync_copy(out_v, out_hbm.at[pl.ds(row0, _CHUNK)])

    inflight = _stage(0, 0)
    for c in range(_NCHUNK):
        for cp in inflight:
            cp.wait()
        if c + 1 < _NCHUNK:
            nxt = _stage(c + 1, (c + 1) % 2)
        else:
            nxt = []
        _compute(c, c % 2)
        inflight = nxt


@jax.jit
def _fm_sc(idx_flat, val_flat, emb_table, w1_flat, bias_vec):
    mesh = plsc.VectorSubcoreMesh(core_axis_name="c", subcore_axis_name="s")
    return pl.kernel(
        _fm_body,
        out_type=jax.ShapeDtypeStruct((_BATCH,), jnp.float32),
        mesh=mesh,
        compiler_params=pltpu.CompilerParams(
            needs_layout_passes=False, use_tc_tiling_on_sc=False),
        scratch_types=[
            pltpu.VMEM((2, _IPC), jnp.int32),
            pltpu.VMEM((2, _IPC), jnp.float32),
            pltpu.VMEM((2, _IPC, _DIM), jnp.float32),
            pltpu.VMEM((2, _IPC), jnp.float32),
            pltpu.VMEM((_CHUNK,), jnp.float32),
            pltpu.VMEM((16,), jnp.float32),
            pltpu.SemaphoreType.DMA,
        ],
    )(idx_flat, val_flat, emb_table, w1_flat, bias_vec)


def kernel(feat_index, feat_value, emb_table, w1, bias):
    idx_flat = feat_index.reshape(-1).astype(jnp.int32)
    val_flat = feat_value.reshape(-1)
    w1_flat = w1.reshape(-1)
    bias_vec = jnp.broadcast_to(jnp.asarray(bias, jnp.float32), (16,))
    return _fm_sc(idx_flat, val_flat, emb_table, w1_flat, bias_vec)
